# initial kernel scaffold (unmeasured)
import functools

import jax
import jax.numpy as jnp
from jax import lax
from jax.experimental import pallas as pl
from jax.experimental.pallas import tpu as pltpu

N_DEV = 32
ROWS = 512
COLS = 256
MAXBIT = 9


def _ring_barrier(sem, me, n_signals):
    def sig(dr, _):
        peer = lax.rem(me + dr, N_DEV)
        pl.semaphore_signal(
            sem, inc=1, device_id=(peer,), device_id_type=pl.DeviceIdType.MESH
        )
        return 0

    lax.fori_loop(1, N_DEV, sig, 0)
    pl.semaphore_wait(sem, n_signals)


def _counts_allgather(counts_row):

    def body(cnt_ref, out_ref, send_sem, recv_sem):
        me = lax.axis_index("i")
        _ring_barrier(pltpu.get_barrier_semaphore(), me, N_DEV - 1)

        out_ref[pl.ds(me, 1), :] = cnt_ref[:, :]

        def send(dr, _):
            peer = lax.rem(me + dr, N_DEV)
            pltpu.make_async_remote_copy(
                src_ref=cnt_ref,
                dst_ref=out_ref.at[pl.ds(me, 1), :],
                send_sem=send_sem,
                recv_sem=recv_sem,
                device_id=(peer,),
                device_id_type=pl.DeviceIdType.MESH,
            ).start()
            return 0

        lax.fori_loop(1, N_DEV, send, 0)

        def wait(dr, _):
            d = pltpu.make_async_remote_copy(
                src_ref=cnt_ref,
                dst_ref=out_ref.at[pl.ds(0, 1), :],
                send_sem=send_sem,
                recv_sem=recv_sem,
                device_id=(me,),
                device_id_type=pl.DeviceIdType.MESH,
            )
            d.wait_send()
            d.wait_recv()
            return 0

        lax.fori_loop(1, N_DEV, wait, 0)

        @functools.partial(pl.run_scoped, sem2=pltpu.SemaphoreType.REGULAR)
        def _(sem2):
            _ring_barrier(sem2, me, N_DEV - 1)

    return pl.pallas_call(
        body,
        out_shape=jax.ShapeDtypeStruct((N_DEV, N_DEV), jnp.int32),
        in_specs=[pl.BlockSpec(memory_space=pltpu.VMEM)],
        out_specs=pl.BlockSpec(memory_space=pltpu.VMEM),
        scratch_shapes=[pltpu.SemaphoreType.DMA, pltpu.SemaphoreType.DMA],
        compiler_params=pltpu.CompilerParams(collective_id=0),
    )(counts_row)


def _a2av(x_sorted, counts, lo, doff, nrem):

    def body(x_ref, cnt_ref, lo_ref, do_ref, nrem_ref, out_ref, send_sem, recv_sem):
        me = lax.axis_index("i")
        _ring_barrier(pltpu.get_barrier_semaphore(), me, N_DEV - 1)

        c_me = cnt_ref[me]
        lo_me = lo_ref[me]
        do_me = do_ref[me]
        for b in range(MAXBIT, -1, -1):
            sz = 1 << b
            base = (c_me >> (b + 1)) << (b + 1)

            @pl.when(((c_me >> b) & 1) == 1)
            def _():
                out_ref[pl.ds(do_me + base, sz), :] = x_ref[pl.ds(lo_me + base, sz), :]

        def send(dr, _):
            r = lax.rem(me + dr, N_DEV)
            c = cnt_ref[r]
            lo_r = lo_ref[r]
            do_r = do_ref[r]
            for b in range(MAXBIT, -1, -1):
                sz = 1 << b
                base = (c >> (b + 1)) << (b + 1)

                @pl.when(((c >> b) & 1) == 1)
                def _():
                    pltpu.make_async_remote_copy(
                        src_ref=x_ref.at[pl.ds(lo_r + base, sz), :],
                        dst_ref=out_ref.at[pl.ds(do_r + base, sz), :],
                        send_sem=send_sem,
                        recv_sem=recv_sem,
                        device_id=(r,),
                        device_id_type=pl.DeviceIdType.MESH,
                    ).start()
            return 0

        lax.fori_loop(1, N_DEV, send, 0)

        n = nrem_ref[0]

        def one_row(i, _):
            d = pltpu.make_async_remote_copy(
                src_ref=x_ref.at[pl.ds(0, 1), :],
                dst_ref=out_ref.at[pl.ds(0, 1), :],
                send_sem=send_sem,
                recv_sem=recv_sem,
                device_id=(me,),
                device_id_type=pl.DeviceIdType.MESH,
            )
            d.wait_send()
            d.wait_recv()
            return 0

        lax.fori_loop(0, n, one_row, 0)

        @functools.partial(pl.run_scoped, sem2=pltpu.SemaphoreType.REGULAR)
        def _(sem2):
            _ring_barrier(sem2, me, N_DEV - 1)

    return pl.pallas_call(
        body,
        out_shape=jax.ShapeDtypeStruct((ROWS, COLS), jnp.bfloat16),
        in_specs=[
            pl.BlockSpec(memory_space=pltpu.VMEM),
            pl.BlockSpec(memory_space=pltpu.SMEM),
            pl.BlockSpec(memory_space=pltpu.SMEM),
            pl.BlockSpec(memory_space=pltpu.SMEM),
            pl.BlockSpec(memory_space=pltpu.SMEM),
        ],
        out_specs=pl.BlockSpec(memory_space=pltpu.VMEM),
        scratch_shapes=[pltpu.SemaphoreType.DMA, pltpu.SemaphoreType.DMA],
        compiler_params=pltpu.CompilerParams(collective_id=1),
    )(x_sorted, counts, lo, doff, nrem)


def kernel(x, dest):
    me = lax.axis_index("i")
    dest = dest.astype(jnp.int32)
    counts = jnp.bincount(dest, length=N_DEV).astype(jnp.int32)

    C = _counts_allgather(counts.reshape(1, N_DEV))

    excl = jnp.cumsum(C, axis=0) - C
    doff = lax.dynamic_index_in_dim(excl, me, axis=0, keepdims=False)
    lo = jnp.cumsum(counts) - counts
    n_own = lax.dynamic_index_in_dim(counts, me, axis=0, keepdims=False)
    nrem = jnp.reshape(jnp.int32(ROWS) - n_own, (1,))

    order = jnp.argsort(dest, stable=True)
    x_sorted = x[order].astype(jnp.bfloat16)

    return _a2av(x_sorted, counts, lo, doff, nrem)


# baseline (device time: 42019 ns/iter reference)
import functools

import jax
import jax.numpy as jnp
from jax import lax
from jax.experimental import pallas as pl
from jax.experimental.pallas import tpu as pltpu

N_DEV = 32
ROWS = 512
COLS = 256
CPAD = 128


def _ring_barrier(sem, me, n_signals):
    def sig(dr, _):
        peer = lax.rem(me + dr, N_DEV)
        pl.semaphore_signal(
            sem, inc=1, device_id=(peer,), device_id_type=pl.DeviceIdType.MESH
        )
        return 0

    lax.fori_loop(1, N_DEV, sig, 0)
    pl.semaphore_wait(sem, n_signals)


def _counts_allgather(dest):

    def body(dest_ref, cnt_hbm_ref, out_ref, cnt_ref, copy_sem, send_sem, recv_sem):
        me = lax.axis_index("i")
        _ring_barrier(pltpu.get_barrier_semaphore(), me, N_DEV - 1)

        dv = dest_ref[:]
        rows_iota = lax.broadcasted_iota(jnp.int32, (CPAD, ROWS), 0)
        cmp = (rows_iota == jnp.broadcast_to(dv[None, :], (CPAD, ROWS))).astype(
            jnp.int32
        )
        cnt_ref[:] = jnp.sum(cmp, axis=1)

        stage = pltpu.make_async_copy(cnt_ref, cnt_hbm_ref, copy_sem)
        stage.start()
        stage.wait()

        my_off = pl.multiple_of(me * CPAD, CPAD)

        def send(dr, _):
            peer = lax.rem(me + dr, N_DEV)
            pltpu.make_async_remote_copy(
                src_ref=cnt_hbm_ref,
                dst_ref=out_ref.at[pl.ds(my_off, CPAD)],
                send_sem=send_sem,
                recv_sem=recv_sem,
                device_id=(peer,),
                device_id_type=pl.DeviceIdType.MESH,
            ).start()
            return 0

        lax.fori_loop(0, N_DEV, send, 0)

        def wait_sends(dr, _):
            pltpu.make_async_remote_copy(
                src_ref=cnt_hbm_ref,
                dst_ref=out_ref.at[pl.ds(0, CPAD)],
                send_sem=send_sem,
                recv_sem=recv_sem,
                device_id=(me,),
                device_id_type=pl.DeviceIdType.MESH,
            ).wait_send()
            return 0

        lax.fori_loop(0, N_DEV, wait_sends, 0)

        pltpu.make_async_remote_copy(
            src_ref=cnt_hbm_ref,
            dst_ref=out_ref,
            send_sem=send_sem,
            recv_sem=recv_sem,
            device_id=(me,),
            device_id_type=pl.DeviceIdType.MESH,
        ).wait_recv()

        @functools.partial(pl.run_scoped, sem2=pltpu.SemaphoreType.REGULAR)
        def _(sem2):
            _ring_barrier(sem2, me, N_DEV - 1)

    _, gathered = pl.pallas_call(
        body,
        out_shape=(
            jax.ShapeDtypeStruct((CPAD,), jnp.int32),
            jax.ShapeDtypeStruct((N_DEV * CPAD,), jnp.int32),
        ),
        in_specs=[pl.BlockSpec(memory_space=pltpu.VMEM)],
        out_specs=(
            pl.BlockSpec(memory_space=pl.ANY),
            pl.BlockSpec(memory_space=pl.ANY),
        ),
        scratch_shapes=[
            pltpu.VMEM((CPAD,), jnp.int32),
            pltpu.SemaphoreType.DMA,
            pltpu.SemaphoreType.DMA,
            pltpu.SemaphoreType.DMA,
        ],
        compiler_params=pltpu.CompilerParams(collective_id=0),
    )(dest)
    return gathered


def _a2av(x_flat, dest, doff):

    def body(x_ref, dest_ref, do_ref, out_ref, seen_ref, send_sem, recv_sem):
        me = lax.axis_index("i")
        _ring_barrier(pltpu.get_barrier_semaphore(), me, N_DEV - 1)

        def zero(r, _):
            seen_ref[r] = 0
            return 0

        lax.fori_loop(0, N_DEV, zero, 0)

        MAXQ = 32

        def send(i, _):
            r = dest_ref[i]
            pos = do_ref[r] + seen_ref[r]
            seen_ref[r] = seen_ref[r] + 1
            src_off = pl.multiple_of(i * COLS, COLS)
            dst_off = pl.multiple_of(pos * COLS, COLS)
            pltpu.make_async_remote_copy(
                src_ref=x_ref.at[pl.ds(src_off, COLS)],
                dst_ref=out_ref.at[pl.ds(dst_off, COLS)],
                send_sem=send_sem,
                recv_sem=recv_sem,
                device_id=(r,),
                device_id_type=pl.DeviceIdType.MESH,
            ).start()

            @pl.when(i >= MAXQ)
            def _():
                pltpu.make_async_remote_copy(
                    src_ref=x_ref.at[pl.ds(0, COLS)],
                    dst_ref=out_ref.at[pl.ds(0, COLS)],
                    send_sem=send_sem,
                    recv_sem=recv_sem,
                    device_id=(me,),
                    device_id_type=pl.DeviceIdType.MESH,
                ).wait_send()

            return 0

        lax.fori_loop(0, ROWS, send, 0)

        pltpu.make_async_remote_copy(
            src_ref=x_ref.at[pl.ds(0, MAXQ * COLS)],
            dst_ref=out_ref.at[pl.ds(0, MAXQ * COLS)],
            send_sem=send_sem,
            recv_sem=recv_sem,
            device_id=(me,),
            device_id_type=pl.DeviceIdType.MESH,
        ).wait_send()
        pltpu.make_async_remote_copy(
            src_ref=x_ref,
            dst_ref=out_ref,
            send_sem=send_sem,
            recv_sem=recv_sem,
            device_id=(me,),
            device_id_type=pl.DeviceIdType.MESH,
        ).wait_recv()

        @functools.partial(pl.run_scoped, sem2=pltpu.SemaphoreType.REGULAR)
        def _(sem2):
            _ring_barrier(sem2, me, N_DEV - 1)

    return pl.pallas_call(
        body,
        out_shape=jax.ShapeDtypeStruct((ROWS * COLS,), jnp.bfloat16),
        in_specs=[
            pl.BlockSpec(memory_space=pl.ANY),
            pl.BlockSpec(memory_space=pltpu.SMEM),
            pl.BlockSpec(memory_space=pltpu.SMEM),
        ],
        out_specs=pl.BlockSpec(memory_space=pl.ANY),
        scratch_shapes=[
            pltpu.SMEM((N_DEV,), jnp.int32),
            pltpu.SemaphoreType.DMA,
            pltpu.SemaphoreType.DMA,
        ],
        compiler_params=pltpu.CompilerParams(collective_id=1),
    )(x_flat, dest, doff)


def kernel(x, dest):
    me = lax.axis_index("i")
    dest = dest.astype(jnp.int32)

    C = _counts_allgather(dest).reshape(N_DEV, CPAD)[:, :N_DEV]

    excl = jnp.cumsum(C, axis=0) - C
    doff = lax.dynamic_index_in_dim(excl, me, axis=0, keepdims=False)

    x_flat = x.astype(jnp.bfloat16).reshape(ROWS * COLS)

    return _a2av(x_flat, dest, doff).reshape(ROWS, COLS)
